# baseline (speedup 1.0000x reference)
import functools

import jax
import jax.numpy as jnp
from jax.experimental import pallas as pl
from jax.experimental.pallas import tpu as pltpu

BB = 8
B = 512
C = 32
H = 32
W = 32
P = H * W


def _conv2x2(x, w4, b):
    bb, h, w, ci = x.shape
    xc = jnp.concatenate(
        [x[:, :-1, :-1, :], x[:, :-1, 1:, :], x[:, 1:, :-1, :], x[:, 1:, 1:, :]],
        axis=-1,
    )
    y = jnp.dot(xc.reshape(bb * (h - 1) * (w - 1), 4 * ci), w4,
                preferred_element_type=jnp.float32) + b
    return y.reshape(bb, h - 1, w - 1, w4.shape[1])


def _maxpool2(x):
    bb, h, w, c = x.shape
    t = x[:, : 2 * (h // 2), : 2 * (w // 2), :]
    t = t.reshape(bb, h // 2, 2, w // 2, 2, c)
    return t.max(axis=4).max(axis=2)


def _nmap_body(pidx_ref,
               m_ref, obs_ref,
               ic1w, ic1b, ic2w, ic2b, ic3w, ic3b,
               rc1w, rc1b, rc2w, rc2b,
               readw, readb, ctxw_s, ctxw_r, ctxb,
               wl1_gm, wl1_s, wl1_r, wl1_c, wl1b, wl2w, wl2b,
               a1r, a1c, a1w, a1b, a2w, a2b,
               c1r, c1c, c1w, c1b, c2w, c2b,
               mnew_ref, lp_ref, val_ref):
    pid = pl.program_id(0)
    (ic1w, ic1b, ic2w, ic2b, ic3w, ic3b,
     rc1w, rc1b, rc2w, rc2b,
     readw, readb, ctxw_s, ctxw_r, ctxb,
     wl1_gm, wl1_s, wl1_r, wl1_c, wl1b, wl2w, wl2b,
     a1r, a1c, a1w, a1b, a2w, a2b,
     c1r, c1c, c1w, c1b, c2w, c2b) = [
        ref[...] for ref in (
            ic1w, ic1b, ic2w, ic2b, ic3w, ic3b,
            rc1w, rc1b, rc2w, rc2b,
            readw, readb, ctxw_s, ctxw_r, ctxb,
            wl1_gm, wl1_s, wl1_r, wl1_c, wl1b, wl2w, wl2b,
            a1r, a1c, a1w, a1b, a2w, a2b,
            c1r, c1c, c1w, c1b, c2w, c2b)]

    xo = obs_ref[...]
    xo = jax.nn.relu(_conv2x2(xo, ic1w, ic1b))
    xo = _maxpool2(xo)
    xo = jax.nn.relu(_conv2x2(xo, ic2w, ic2b))
    xo = jax.nn.relu(_conv2x2(xo, ic3w, ic3b))
    s = xo.reshape(BB, 5 * 5 * 64)

    m = m_ref[...]
    mf = m.reshape(BB, C, P)
    mcl = jnp.swapaxes(mf, 1, 2).reshape(BB, H, W, C)
    t = jax.nn.relu(_conv2x2(mcl, rc1w, rc1b))
    t = _maxpool2(t)
    t = jax.nn.relu(_conv2x2(t, rc2w, rc2b))
    r = jnp.dot(t.reshape(BB, 14 * 14 * 64), readw,
                preferred_element_type=jnp.float32) + readb

    q = (jnp.dot(s, ctxw_s, preferred_element_type=jnp.float32)
         + jnp.dot(r, ctxw_r, preferred_element_type=jnp.float32) + ctxb)

    scores = jnp.sum(mf * q[:, :, None], axis=1)
    smax = jnp.max(scores, axis=-1, keepdims=True)
    e = jnp.exp(scores - smax)
    attn = e / jnp.sum(e, axis=-1, keepdims=True)
    cvec = jnp.sum(mf * attn[:, None, :], axis=-1)

    lane = jax.lax.broadcasted_iota(jnp.int32, (1, P), 1)
    rows = [(lane == pidx_ref[pid * BB + j]).astype(jnp.float32) for j in range(BB)]
    mask = jnp.concatenate(rows, axis=0)
    gm = jnp.sum(mf * mask[:, None, :], axis=-1)

    t2 = jax.nn.relu(jnp.dot(gm, wl1_gm, preferred_element_type=jnp.float32)
                     + jnp.dot(s, wl1_s, preferred_element_type=jnp.float32)
                     + jnp.dot(r, wl1_r, preferred_element_type=jnp.float32)
                     + jnp.dot(cvec, wl1_c, preferred_element_type=jnp.float32)
                     + wl1b)
    wv = jnp.dot(t2, wl2w, preferred_element_type=jnp.float32) + wl2b

    mnew = mf + mask[:, None, :] * (wv[:, :, None] - mf)
    mnew_ref[...] = mnew.reshape(BB, C, H, W)

    h1 = jnp.tanh(jnp.dot(r, a1r, preferred_element_type=jnp.float32)
                  + jnp.dot(cvec, a1c, preferred_element_type=jnp.float32)
                  + jnp.dot(wv, a1w, preferred_element_type=jnp.float32) + a1b)
    logits = jnp.dot(h1, a2w, preferred_element_type=jnp.float32) + a2b
    lmax = jnp.max(logits, axis=-1, keepdims=True)
    lse = jnp.log(jnp.sum(jnp.exp(logits - lmax), axis=-1, keepdims=True)) + lmax
    lp_ref[...] = logits - lse

    h2 = jnp.tanh(jnp.dot(r, c1r, preferred_element_type=jnp.float32)
                  + jnp.dot(cvec, c1c, preferred_element_type=jnp.float32)
                  + jnp.dot(wv, c1w, preferred_element_type=jnp.float32) + c1b)
    val_ref[...] = jnp.dot(h2, c2w, preferred_element_type=jnp.float32) + c2b


def _pack_conv(w):
    return jnp.transpose(w, (2, 3, 1, 0)).reshape(-1, w.shape[0])


def _perm_hwc(w, c, h, ww):
    return jnp.transpose(w.reshape(w.shape[0], c, h, ww), (2, 3, 1, 0)).reshape(-1, w.shape[0])


@jax.jit
def kernel(M, obs_image, memory, pos, params):
    p = params
    p_idx = (W * pos[:, 0] + pos[:, 1]).astype(jnp.int32)

    row2 = lambda b: b.reshape(1, -1)
    weights = (
        _pack_conv(p['ic1_w']), row2(p['ic1_b']),
        _pack_conv(p['ic2_w']), row2(p['ic2_b']),
        _pack_conv(p['ic3_w']), row2(p['ic3_b']),
        _pack_conv(p['rc1_w']), row2(p['rc1_b']),
        _pack_conv(p['rc2_w']), row2(p['rc2_b']),
        _perm_hwc(p['read_w'], 64, 14, 14), row2(p['read_b']),
        _perm_hwc(p['ctx_w'][:, :1600], 64, 5, 5), p['ctx_w'][:, 1600:].T, row2(p['ctx_b']),
        p['wl1_w'][:, :32].T, _perm_hwc(p['wl1_w'][:, 32:1632], 64, 5, 5),
        p['wl1_w'][:, 1632:1664].T, p['wl1_w'][:, 1664:].T, row2(p['wl1_b']),
        p['wl2_w'].T, row2(p['wl2_b']),
        p['act1_w'][:, :32].T, p['act1_w'][:, 32:64].T, p['act1_w'][:, 64:].T, row2(p['act1_b']),
        p['act2_w'].T, row2(p['act2_b']),
        p['cr1_w'][:, :32].T, p['cr1_w'][:, 32:64].T, p['cr1_w'][:, 64:].T, row2(p['cr1_b']),
        p['cr2_w'].T, row2(p['cr2_b']),
    )

    wspecs = [pl.BlockSpec(w.shape, functools.partial(lambda nd, i, *_: (0,) * nd, w.ndim))
              for w in weights]

    grid_spec = pltpu.PrefetchScalarGridSpec(
        num_scalar_prefetch=1,
        grid=(B // BB,),
        in_specs=[
            pl.BlockSpec((BB, C, H, W), lambda i, *_: (i, 0, 0, 0)),
            pl.BlockSpec((BB, 15, 15, 3), lambda i, *_: (i, 0, 0, 0)),
            *wspecs,
        ],
        out_specs=[
            pl.BlockSpec((BB, C, H, W), lambda i, *_: (i, 0, 0, 0)),
            pl.BlockSpec((BB, 7), lambda i, *_: (i, 0)),
            pl.BlockSpec((BB, 1), lambda i, *_: (i, 0)),
        ],
    )

    m_new, log_probs, value = pl.pallas_call(
        _nmap_body,
        grid_spec=grid_spec,
        out_shape=[
            jax.ShapeDtypeStruct((B, C, H, W), jnp.float32),
            jax.ShapeDtypeStruct((B, 7), jnp.float32),
            jax.ShapeDtypeStruct((B, 1), jnp.float32),
        ],
        compiler_params=pltpu.CompilerParams(
            dimension_semantics=("parallel",),
        ),
    )(p_idx, M, obs_image, *weights)

    return log_probs, value[:, 0], memory, m_new
